# baseline (speedup 1.0000x reference)
import functools

import jax
import jax.numpy as jnp
from jax.experimental import pallas as pl
from jax.experimental.pallas import tpu as pltpu

_CI = 64
_CO = 128
_S = 24
_K = 3
_KCAT = _K * _CI
_BD = 6
_NDB = _S // _BD
_PB = _BD * _S * _S


def _p1_conv_stats(x_ref, w_ref, conv_ref, stats_ref, xc_ref):
    j = pl.program_id(1)

    @pl.when(j == 0)
    def _init():
        xc_ref[...] = jnp.zeros_like(xc_ref)
        xc_ref[1:_S + 1, 1:_S + 1, 1:_S, 0:_CI] = x_ref[:, :, 0:_S - 1, :]
        xc_ref[1:_S + 1, 1:_S + 1, :, _CI:2 * _CI] = x_ref[:, :, :, :]
        xc_ref[1:_S + 1, 1:_S + 1, 0:_S - 1, 2 * _CI:3 * _CI] = x_ref[:, :, 1:_S, :]
        stats_ref[...] = jnp.zeros_like(stats_ref)

    d0 = j * _BD
    acc = jnp.zeros((_PB, _CO), jnp.float32)
    for a in range(_K):
        for b in range(_K):
            lhs = xc_ref[pl.ds(d0 + a, _BD), pl.ds(b, _S), :, :]
            acc = acc + jnp.dot(lhs.reshape(_PB, _KCAT),
                                w_ref[_K * a + b],
                                preferred_element_type=jnp.float32)
    conv_ref[...] = acc.astype(jnp.bfloat16).T
    stats_ref[0:1, :] += jnp.sum(acc, axis=0, keepdims=True)
    stats_ref[1:2, :] += jnp.sum(acc * acc, axis=0, keepdims=True)


def _p2_bn_relu_t(conv_ref, scale_ref, shift_ref, o_ref):
    scale = scale_ref[...].reshape(_CO, 1)
    shift = shift_ref[...].reshape(_CO, 1)
    y = conv_ref[...].astype(jnp.float32) * scale + shift
    o_ref[...] = jnp.maximum(y, 0.0)


def kernel(x, weight, gamma, beta):
    N = x.shape[0]
    eps = 1e-5
    P = _S * _S * _S

    xl = jnp.transpose(x, (0, 2, 3, 4, 1)).astype(jnp.bfloat16)

    wt = jnp.transpose(weight, (2, 3, 4, 1, 0))
    wt = wt.reshape(_K * _K, _KCAT, _CO).astype(jnp.bfloat16)

    conv, stats = pl.pallas_call(
        _p1_conv_stats,
        out_shape=[
            jax.ShapeDtypeStruct((N, _CO, P), jnp.bfloat16),
            jax.ShapeDtypeStruct((N, 2, _CO), jnp.float32),
        ],
        grid=(N, _NDB),
        in_specs=[
            pl.BlockSpec((None, _S, _S, _S, _CI),
                         lambda n, j: (n, 0, 0, 0, 0)),
            pl.BlockSpec((_K * _K, _KCAT, _CO), lambda n, j: (0, 0, 0)),
        ],
        out_specs=[
            pl.BlockSpec((None, _CO, _PB), lambda n, j: (n, 0, j)),
            pl.BlockSpec((None, 2, _CO), lambda n, j: (n, 0, 0)),
        ],
        scratch_shapes=[
            pltpu.VMEM((_S + 2, _S + 2, _S, _KCAT), jnp.bfloat16),
        ],
        compiler_params=pltpu.CompilerParams(
            dimension_semantics=("parallel", "arbitrary")),
    )(xl, wt)

    M = N * P
    sums = jnp.sum(stats, axis=0)
    mean = sums[0] / M
    var = sums[1] / M - mean * mean
    scale = gamma.astype(jnp.float32) * jax.lax.rsqrt(var + eps)
    shift = beta.astype(jnp.float32) - mean * scale

    out_flat = pl.pallas_call(
        _p2_bn_relu_t,
        out_shape=jax.ShapeDtypeStruct((N, _CO, P), jnp.float32),
        grid=(N,),
        in_specs=[
            pl.BlockSpec((None, _CO, P), lambda n: (n, 0, 0)),
            pl.BlockSpec((1, _CO), lambda n: (0, 0)),
            pl.BlockSpec((1, _CO), lambda n: (0, 0)),
        ],
        out_specs=pl.BlockSpec((None, _CO, P), lambda n: (n, 0, 0)),
        compiler_params=pltpu.CompilerParams(
            dimension_semantics=("parallel",)),
    )(conv, scale.reshape(1, _CO), shift.reshape(1, _CO))

    return out_flat.reshape(N, _CO, _S, _S, _S)


# speedup vs baseline: 1.1809x; 1.1809x over previous
import functools

import jax
import jax.numpy as jnp
from jax.experimental import pallas as pl
from jax.experimental.pallas import tpu as pltpu

_CI = 64
_CO = 128
_S = 24
_K = 3
_KCAT = _K * _CI
_BD = 6
_NDB = _S // _BD
_PB = _BD * _S * _S


def _p1_conv_stats(x_ref, w_ref, conv_ref, stats_ref, xc_ref):
    j = pl.program_id(1)

    @pl.when(j == 0)
    def _init():
        xc_ref[...] = jnp.zeros_like(xc_ref)
        xc_ref[1:_S + 1, 1:_S + 1, 1:_S, 0:_CI] = x_ref[:, :, 0:_S - 1, :]
        xc_ref[1:_S + 1, 1:_S + 1, :, _CI:2 * _CI] = x_ref[:, :, :, :]
        xc_ref[1:_S + 1, 1:_S + 1, 0:_S - 1, 2 * _CI:3 * _CI] = x_ref[:, :, 1:_S, :]
        stats_ref[...] = jnp.zeros_like(stats_ref)

    d0 = j * _BD
    acc = jnp.zeros((_PB, _CO), jnp.float32)
    for a in range(_K):
        for b in range(_K):
            lhs = xc_ref[pl.ds(d0 + a, _BD), pl.ds(b, _S), :, :]
            acc = acc + jnp.dot(lhs.reshape(_PB, _KCAT),
                                w_ref[_K * a + b],
                                preferred_element_type=jnp.float32)
    conv_ref[...] = acc.astype(jnp.bfloat16)
    stats_ref[0:1, :] += jnp.sum(acc, axis=0, keepdims=True)
    stats_ref[1:2, :] += jnp.sum(acc * acc, axis=0, keepdims=True)


def _p2_bn_relu_t(conv_ref, scale_ref, shift_ref, o_ref):
    y = conv_ref[...].astype(jnp.float32) * scale_ref[...] + shift_ref[...]
    o_ref[...] = jnp.maximum(y, 0.0)


def kernel(x, weight, gamma, beta):
    N = x.shape[0]
    eps = 1e-5
    P = _S * _S * _S

    xl = jnp.transpose(x, (0, 2, 3, 4, 1)).astype(jnp.bfloat16)

    wt = jnp.transpose(weight, (2, 3, 4, 1, 0))
    wt = wt.reshape(_K * _K, _KCAT, _CO).astype(jnp.bfloat16)

    conv, stats = pl.pallas_call(
        _p1_conv_stats,
        out_shape=[
            jax.ShapeDtypeStruct((N, P, _CO), jnp.bfloat16),
            jax.ShapeDtypeStruct((N, 2, _CO), jnp.float32),
        ],
        grid=(N, _NDB),
        in_specs=[
            pl.BlockSpec((None, _S, _S, _S, _CI),
                         lambda n, j: (n, 0, 0, 0, 0)),
            pl.BlockSpec((_K * _K, _KCAT, _CO), lambda n, j: (0, 0, 0)),
        ],
        out_specs=[
            pl.BlockSpec((None, _PB, _CO), lambda n, j: (n, j, 0)),
            pl.BlockSpec((None, 2, _CO), lambda n, j: (n, 0, 0)),
        ],
        scratch_shapes=[
            pltpu.VMEM((_S + 2, _S + 2, _S, _KCAT), jnp.bfloat16),
        ],
        compiler_params=pltpu.CompilerParams(
            dimension_semantics=("parallel", "arbitrary")),
    )(xl, wt)

    M = N * P
    sums = jnp.sum(stats, axis=0)
    mean = sums[0] / M
    var = sums[1] / M - mean * mean
    scale = gamma.astype(jnp.float32) * jax.lax.rsqrt(var + eps)
    shift = beta.astype(jnp.float32) - mean * scale

    out_flat = pl.pallas_call(
        _p2_bn_relu_t,
        out_shape=jax.ShapeDtypeStruct((N, P, _CO), jnp.float32),
        grid=(N, _NDB),
        in_specs=[
            pl.BlockSpec((None, _PB, _CO), lambda n, j: (n, j, 0)),
            pl.BlockSpec((1, _CO), lambda n, j: (0, 0)),
            pl.BlockSpec((1, _CO), lambda n, j: (0, 0)),
        ],
        out_specs=pl.BlockSpec((None, _PB, _CO), lambda n, j: (n, j, 0)),
        compiler_params=pltpu.CompilerParams(
            dimension_semantics=("parallel", "parallel")),
    )(conv, scale.reshape(1, _CO), shift.reshape(1, _CO))

    out = out_flat.reshape(N, _S, _S, _S, _CO)
    return jnp.transpose(out, (0, 4, 1, 2, 3))


# speedup vs baseline: 1.1888x; 1.0067x over previous
import functools

import jax
import jax.numpy as jnp
from jax.experimental import pallas as pl
from jax.experimental.pallas import tpu as pltpu

_CI = 64
_CO = 128
_S = 24
_K = 3
_KCAT = _K * _CI
_BD = 6
_NDB = _S // _BD
_PB = _BD * _S * _S


def _p1_conv_stats(x_ref, w_ref, conv_ref, stats_ref, xc_ref):
    j = pl.program_id(1)

    @pl.when(j == 0)
    def _init():
        xc_ref[...] = jnp.zeros_like(xc_ref)
        xc_ref[1:_S + 1, 1:_S + 1, 1:_S, 0:_CI] = x_ref[:, :, 0:_S - 1, :]
        xc_ref[1:_S + 1, 1:_S + 1, :, _CI:2 * _CI] = x_ref[:, :, :, :]
        xc_ref[1:_S + 1, 1:_S + 1, 0:_S - 1, 2 * _CI:3 * _CI] = x_ref[:, :, 1:_S, :]
        stats_ref[...] = jnp.zeros_like(stats_ref)

    d0 = j * _BD
    _PS = _S * _S
    s_tot = jnp.zeros((1, _CO), jnp.float32)
    q_tot = jnp.zeros((1, _CO), jnp.float32)
    for d in range(_BD):
        acc = jnp.zeros((_PS, _CO), jnp.float32)
        for a in range(_K):
            for b in range(_K):
                lhs = xc_ref[pl.ds(d0 + d + a, 1), pl.ds(b, _S), :, :]
                acc = acc + jnp.dot(lhs.reshape(_PS, _KCAT),
                                    w_ref[_K * a + b],
                                    preferred_element_type=jnp.float32)
        conv_ref[pl.ds(d * _PS, _PS), :] = acc.astype(jnp.bfloat16)
        s_tot = s_tot + jnp.sum(acc, axis=0, keepdims=True)
        q_tot = q_tot + jnp.sum(acc * acc, axis=0, keepdims=True)
    stats_ref[0:1, :] += s_tot
    stats_ref[1:2, :] += q_tot


def _p2_bn_relu_t(conv_ref, scale_ref, shift_ref, o_ref):
    y = conv_ref[...].astype(jnp.float32) * scale_ref[...] + shift_ref[...]
    o_ref[...] = jnp.maximum(y, 0.0)


def kernel(x, weight, gamma, beta):
    N = x.shape[0]
    eps = 1e-5
    P = _S * _S * _S

    xl = jnp.transpose(x, (0, 2, 3, 4, 1)).astype(jnp.bfloat16)

    wt = jnp.transpose(weight, (2, 3, 4, 1, 0))
    wt = wt.reshape(_K * _K, _KCAT, _CO).astype(jnp.bfloat16)

    conv, stats = pl.pallas_call(
        _p1_conv_stats,
        out_shape=[
            jax.ShapeDtypeStruct((N, P, _CO), jnp.bfloat16),
            jax.ShapeDtypeStruct((N, 2, _CO), jnp.float32),
        ],
        grid=(N, _NDB),
        in_specs=[
            pl.BlockSpec((None, _S, _S, _S, _CI),
                         lambda n, j: (n, 0, 0, 0, 0)),
            pl.BlockSpec((_K * _K, _KCAT, _CO), lambda n, j: (0, 0, 0)),
        ],
        out_specs=[
            pl.BlockSpec((None, _PB, _CO), lambda n, j: (n, j, 0)),
            pl.BlockSpec((None, 2, _CO), lambda n, j: (n, 0, 0)),
        ],
        scratch_shapes=[
            pltpu.VMEM((_S + 2, _S + 2, _S, _KCAT), jnp.bfloat16),
        ],
        compiler_params=pltpu.CompilerParams(
            dimension_semantics=("parallel", "arbitrary")),
    )(xl, wt)

    M = N * P
    sums = jnp.sum(stats, axis=0)
    mean = sums[0] / M
    var = sums[1] / M - mean * mean
    scale = gamma.astype(jnp.float32) * jax.lax.rsqrt(var + eps)
    shift = beta.astype(jnp.float32) - mean * scale

    out_flat = pl.pallas_call(
        _p2_bn_relu_t,
        out_shape=jax.ShapeDtypeStruct((N, P, _CO), jnp.float32),
        grid=(N, _NDB),
        in_specs=[
            pl.BlockSpec((None, _PB, _CO), lambda n, j: (n, j, 0)),
            pl.BlockSpec((1, _CO), lambda n, j: (0, 0)),
            pl.BlockSpec((1, _CO), lambda n, j: (0, 0)),
        ],
        out_specs=pl.BlockSpec((None, _PB, _CO), lambda n, j: (n, j, 0)),
        compiler_params=pltpu.CompilerParams(
            dimension_semantics=("parallel", "parallel")),
    )(conv, scale.reshape(1, _CO), shift.reshape(1, _CO))

    out = out_flat.reshape(N, _S, _S, _S, _CO)
    return jnp.transpose(out, (0, 4, 1, 2, 3))


# speedup vs baseline: 1.4201x; 1.1946x over previous
import functools

import jax
import jax.numpy as jnp
from jax.experimental import pallas as pl
from jax.experimental.pallas import tpu as pltpu

_CI = 64
_CO = 128
_S = 24
_K = 3
_KCAT = _K * _CI
_KG = 256
_KF = _K * _KG
_NS = _K * _CO
_PS = _S * _S


def _p1_conv_stats(x_ref, w_ref, conv_ref, stats_ref, xc_ref, xc2_ref):
    xc_ref[...] = jnp.zeros_like(xc_ref)
    xc_ref[1:_S + 1, 1:_S + 1, 1:_S, 0:_CI] = x_ref[:, :, 0:_S - 1, :]
    xc_ref[1:_S + 1, 1:_S + 1, :, _CI:2 * _CI] = x_ref[:, :, :, :]
    xc_ref[1:_S + 1, 1:_S + 1, 0:_S - 1, 2 * _CI:3 * _CI] = x_ref[:, :, 1:_S, :]
    for b in range(_K):
        xc2_ref[:, :, :, b * _KG:b * _KG + _KCAT] = xc_ref[:, b:b + _S, :, :]
        xc2_ref[:, :, :, b * _KG + _KCAT:(b + 1) * _KG] = jnp.zeros(
            (_S + 2, _S, _S, _KG - _KCAT), jnp.bfloat16)

    s_tot = jnp.zeros((1, _CO), jnp.float32)
    q_tot = jnp.zeros((1, _CO), jnp.float32)
    accs = {}
    for t in range(_S + 2):
        lhs = xc2_ref[t].reshape(_PS, _KF)
        y = jnp.dot(lhs, w_ref[...], preferred_element_type=jnp.float32)
        for a in range(_K):
            d = t - a
            if 0 <= d < _S:
                chunk = y[:, a * _CO:(a + 1) * _CO]
                accs[d] = chunk if a == 0 else accs[d] + chunk
        d_done = t - (_K - 1)
        if 0 <= d_done < _S:
            accf = accs.pop(d_done)
            conv_ref[pl.ds(d_done * _PS, _PS), :] = accf.astype(jnp.bfloat16)
            s_tot = s_tot + jnp.sum(accf, axis=0, keepdims=True)
            q_tot = q_tot + jnp.sum(accf * accf, axis=0, keepdims=True)
    stats_ref[0:1, :] = s_tot
    stats_ref[1:2, :] = q_tot


def _p2_bn_relu(conv_ref, scale_ref, shift_ref, o_ref):
    y = conv_ref[...].astype(jnp.float32) * scale_ref[...] + shift_ref[...]
    o_ref[...] = jnp.maximum(y, 0.0)


def kernel(x, weight, gamma, beta):
    N = x.shape[0]
    eps = 1e-5
    P = _S * _S * _S

    xl = jnp.transpose(x, (0, 2, 3, 4, 1)).astype(jnp.bfloat16)

    w5 = jnp.transpose(weight, (3, 4, 1, 2, 0))
    w5 = w5.reshape(_K, _KCAT, _K, _CO)
    w5 = jnp.pad(w5, ((0, 0), (0, _KG - _KCAT), (0, 0), (0, 0)))
    wt = w5.reshape(_KF, _NS).astype(jnp.bfloat16)

    conv, stats = pl.pallas_call(
        _p1_conv_stats,
        out_shape=[
            jax.ShapeDtypeStruct((N, P, _CO), jnp.bfloat16),
            jax.ShapeDtypeStruct((N, 2, _CO), jnp.float32),
        ],
        grid=(N,),
        in_specs=[
            pl.BlockSpec((None, _S, _S, _S, _CI), lambda n: (n, 0, 0, 0, 0)),
            pl.BlockSpec((_KF, _NS), lambda n: (0, 0)),
        ],
        out_specs=[
            pl.BlockSpec((None, P, _CO), lambda n: (n, 0, 0)),
            pl.BlockSpec((None, 2, _CO), lambda n: (n, 0, 0)),
        ],
        scratch_shapes=[
            pltpu.VMEM((_S + 2, _S + 2, _S, _KCAT), jnp.bfloat16),
            pltpu.VMEM((_S + 2, _S, _S, _KF), jnp.bfloat16),
        ],
        compiler_params=pltpu.CompilerParams(
            dimension_semantics=("parallel",)),
    )(xl, wt)

    M = N * P
    sums = jnp.sum(stats, axis=0)
    mean = sums[0] / M
    var = sums[1] / M - mean * mean
    scale = gamma.astype(jnp.float32) * jax.lax.rsqrt(var + eps)
    shift = beta.astype(jnp.float32) - mean * scale

    _PB = P // 4
    out_flat = pl.pallas_call(
        _p2_bn_relu,
        out_shape=jax.ShapeDtypeStruct((N, P, _CO), jnp.float32),
        grid=(N, 4),
        in_specs=[
            pl.BlockSpec((None, _PB, _CO), lambda n, j: (n, j, 0)),
            pl.BlockSpec((1, _CO), lambda n, j: (0, 0)),
            pl.BlockSpec((1, _CO), lambda n, j: (0, 0)),
        ],
        out_specs=pl.BlockSpec((None, _PB, _CO), lambda n, j: (n, j, 0)),
        compiler_params=pltpu.CompilerParams(
            dimension_semantics=("parallel", "parallel")),
    )(conv, scale.reshape(1, _CO), shift.reshape(1, _CO))

    out = out_flat.reshape(N, _S, _S, _S, _CO)
    return jnp.transpose(out, (0, 4, 1, 2, 3))
